# baseline (speedup 1.0000x reference)
import functools

import jax
import jax.numpy as jnp
from jax import lax
from jax.experimental import pallas as pl
from jax.experimental.pallas import tpu as pltpu
from jax.experimental.pallas import tpu_sc as plsc

NC = 2
NS = 16
NW = NC * NS
L = 16

R = 32768
C = 1000
B = 4096
RPW = R // NW
CHUNK_ROWS = 16
NCHUNKS = RPW // CHUNK_ROWS
CHUNK_WORDS = CHUNK_ROWS * C
BUF_WORDS = CHUNK_WORDS + L
LBL_WORDS = RPW + L

_mesh = plsc.VectorSubcoreMesh(
    core_axis_name="c", subcore_axis_name="s", num_cores=NC, num_subcores=NS
)


@functools.partial(
    pl.kernel,
    out_type=jax.ShapeDtypeStruct((NW, L), jnp.float32),
    mesh=_mesh,
    compiler_params=pltpu.CompilerParams(needs_layout_passes=False),
    scratch_types=[
        pltpu.VMEM((BUF_WORDS,), jnp.float32),
        pltpu.VMEM((LBL_WORDS,), jnp.int32),
        pltpu.VMEM((L,), jnp.float32),
    ],
)
def _loss_partials(pred_hbm, label_hbm, out_hbm, buf, lbl, res):
    cid = lax.axis_index("c")
    sid = lax.axis_index("s")
    wid = sid * NC + cid
    b_base = lax.rem(wid, B // RPW) * RPW
    pltpu.sync_copy(label_hbm.at[pl.ds(b_base, RPW)], lbl.at[pl.ds(0, RPW)])

    lane = lax.iota(jnp.int32, L)
    ninf = jnp.full((L,), -jnp.inf, jnp.float32)
    zero = jnp.zeros((L,), jnp.float32)
    row0 = wid * RPW

    def chunk_body(c, acc):
        pltpu.sync_copy(
            pred_hbm.at[pl.ds((row0 + c * CHUNK_ROWS) * C, CHUNK_WORDS)],
            buf.at[pl.ds(0, CHUNK_WORDS)],
        )

        def row_body(j, acc):
            base = j * C
            lblv = plsc.load_gather(
                lbl, [jnp.full((L,), c * CHUNK_ROWS + j, jnp.int32)]
            )
            addr = lblv + base
            v = plsc.load_gather(buf, [addr])
            plsc.store_scatter(
                buf, [addr], jnp.full((L,), -1e10, jnp.float32), mask=lane < 1
            )

            def vmax(i, m):
                return jnp.maximum(m, buf[pl.ds(base + i * L, L)])

            m = lax.fori_loop(0, C // L, vmax, ninf)
            tail = buf[pl.ds(base + (C // L) * L, L)]
            m = jnp.maximum(m, jnp.where(lane < (C % L), tail, ninf))
            fnym = jnp.max(m)
            loss0 = jnp.maximum(2.0 - v, 0.0) + jnp.maximum(1.0 + fnym, 0.0)
            return acc + jnp.where(lane < 1, loss0, zero)

        return lax.fori_loop(0, CHUNK_ROWS, row_body, acc)

    acc = lax.fori_loop(0, NCHUNKS, chunk_body, jnp.zeros((L,), jnp.float32))
    res[...] = acc
    pltpu.sync_copy(res, out_hbm.at[wid])


def kernel(prediction, label):
    pred_flat = prediction.reshape(-1)
    partials = _loss_partials(pred_flat, label)
    return jnp.sum(partials) / jnp.float32(R)


# speedup vs baseline: 1.8835x; 1.8835x over previous
import functools

import jax
import jax.numpy as jnp
from jax import lax
from jax.experimental import pallas as pl
from jax.experimental.pallas import tpu as pltpu
from jax.experimental.pallas import tpu_sc as plsc

NC = 2
NS = 16
NW = NC * NS
L = 16

R = 32768
C = 1000
B = 4096
RPW = R // NW
CHUNK_ROWS = 16
NCHUNKS = RPW // CHUNK_ROWS
CHUNK_WORDS = CHUNK_ROWS * C
BUF_WORDS = CHUNK_WORDS + L
LBL_WORDS = RPW + L
NBUF = 2
NVREG = C // L
CTAIL = C - NVREG * L

_mesh = plsc.VectorSubcoreMesh(
    core_axis_name="c", subcore_axis_name="s", num_cores=NC, num_subcores=NS
)


@functools.partial(
    pl.kernel,
    out_type=jax.ShapeDtypeStruct((NW, L), jnp.float32),
    mesh=_mesh,
    compiler_params=pltpu.CompilerParams(needs_layout_passes=False),
    scratch_types=[
        pltpu.VMEM((BUF_WORDS,), jnp.float32),
        pltpu.VMEM((BUF_WORDS,), jnp.float32),
        pltpu.VMEM((LBL_WORDS,), jnp.int32),
        pltpu.VMEM((L,), jnp.float32),
        pltpu.SemaphoreType.DMA,
        pltpu.SemaphoreType.DMA,
    ],
)
def _loss_partials(pred_hbm, label_hbm, out_hbm, buf0, buf1, lbl, res, sem0, sem1):
    cid = lax.axis_index("c")
    sid = lax.axis_index("s")
    wid = sid * NC + cid
    b_base = lax.rem(wid, B // RPW) * RPW
    pltpu.sync_copy(label_hbm.at[pl.ds(b_base, RPW)], lbl.at[pl.ds(0, RPW)])

    lane = lax.iota(jnp.int32, L)
    ninf = jnp.full((L,), -jnp.inf, jnp.float32)
    zero = jnp.zeros((L,), jnp.float32)
    row0 = wid * RPW
    sems = (sem0, sem1)
    bufs = (buf0, buf1)

    def chunk_src(c):
        return pred_hbm.at[pl.ds((row0 + c * CHUNK_ROWS) * C, CHUNK_WORDS)]

    def chunk_dst(b):
        return bufs[b].at[pl.ds(0, CHUNK_WORDS)]

    for b in range(NBUF):
        pltpu.async_copy(chunk_src(b), chunk_dst(b), sems[b])

    def compute_chunk(c, b, acc):
        buf = bufs[b]

        def row_body(j, acc):
            base = j * C
            lblv = plsc.load_gather(
                lbl, [jnp.full((L,), c * CHUNK_ROWS + j, jnp.int32)]
            )
            addr = lblv + base
            fy = plsc.load_gather(buf, [addr])
            plsc.store_scatter(
                buf, [addr], jnp.full((L,), -1e10, jnp.float32), mask=lane < 1
            )
            vals = [buf[pl.ds(base + k * L, L)] for k in range(NVREG)]
            vals.append(
                jnp.where(lane < CTAIL, buf[pl.ds(base + NVREG * L, L)], ninf)
            )
            while len(vals) > 1:
                nxt = [
                    jnp.maximum(a, b2) for a, b2 in zip(vals[0::2], vals[1::2])
                ]
                if len(vals) % 2:
                    nxt.append(vals[-1])
                vals = nxt
            fnym = jnp.max(vals[0])
            loss0 = jnp.maximum(2.0 - fy, 0.0) + jnp.maximum(1.0 + fnym, 0.0)
            return acc + jnp.where(lane < 1, loss0, zero)

        return lax.fori_loop(0, CHUNK_ROWS, row_body, acc)

    def ring_body(g, acc):
        for b in range(NBUF):
            c = g * NBUF + b
            pltpu.make_async_copy(chunk_src(c), chunk_dst(b), sems[b]).wait()
            acc = compute_chunk(c, b, acc)

            @pl.when(g < NCHUNKS // NBUF - 1)
            def _():
                pltpu.async_copy(chunk_src(c + NBUF), chunk_dst(b), sems[b])

        return acc

    acc = lax.fori_loop(
        0, NCHUNKS // NBUF, ring_body, jnp.zeros((L,), jnp.float32)
    )
    res[...] = acc
    pltpu.sync_copy(res, out_hbm.at[wid])


def kernel(prediction, label):
    pred_flat = prediction.reshape(-1)
    partials = _loss_partials(pred_flat, label)
    return jnp.sum(partials) / jnp.float32(R)


# speedup vs baseline: 2.0112x; 1.0678x over previous
import functools

import jax
import jax.numpy as jnp
from jax import lax
from jax.experimental import pallas as pl
from jax.experimental.pallas import tpu as pltpu
from jax.experimental.pallas import tpu_sc as plsc

NC = 2
NS = 16
NW = NC * NS
L = 16

R = 32768
C = 1000
B = 4096
RPW = R // NW
CHUNK_ROWS = 16
NCHUNKS = RPW // CHUNK_ROWS
CHUNK_WORDS = CHUNK_ROWS * C
BUF_WORDS = CHUNK_WORDS + L
LBL_WORDS = RPW + L
NBUF = 4
NVREG = C // L
CTAIL = C - NVREG * L

_mesh = plsc.VectorSubcoreMesh(
    core_axis_name="c", subcore_axis_name="s", num_cores=NC, num_subcores=NS
)


@functools.partial(
    pl.kernel,
    out_type=jax.ShapeDtypeStruct((NW, L), jnp.float32),
    mesh=_mesh,
    compiler_params=pltpu.CompilerParams(needs_layout_passes=False),
    scratch_types=[
        pltpu.VMEM((BUF_WORDS,), jnp.float32),
        pltpu.VMEM((BUF_WORDS,), jnp.float32),
        pltpu.VMEM((BUF_WORDS,), jnp.float32),
        pltpu.VMEM((BUF_WORDS,), jnp.float32),
        pltpu.VMEM((LBL_WORDS,), jnp.int32),
        pltpu.VMEM((L,), jnp.float32),
        pltpu.SemaphoreType.DMA,
        pltpu.SemaphoreType.DMA,
        pltpu.SemaphoreType.DMA,
        pltpu.SemaphoreType.DMA,
    ],
)
def _loss_partials(pred_hbm, label_hbm, out_hbm, buf0, buf1, buf2, buf3, lbl, res, sem0, sem1, sem2, sem3):
    cid = lax.axis_index("c")
    sid = lax.axis_index("s")
    wid = sid * NC + cid
    b_base = lax.rem(wid, B // RPW) * RPW
    pltpu.sync_copy(label_hbm.at[pl.ds(b_base, RPW)], lbl.at[pl.ds(0, RPW)])

    lane = lax.iota(jnp.int32, L)
    ninf = jnp.full((L,), -jnp.inf, jnp.float32)
    zero = jnp.zeros((L,), jnp.float32)
    row0 = wid * RPW
    sems = (sem0, sem1, sem2, sem3)
    bufs = (buf0, buf1, buf2, buf3)

    def chunk_src(c):
        return pred_hbm.at[pl.ds((row0 + c * CHUNK_ROWS) * C, CHUNK_WORDS)]

    def chunk_dst(b):
        return bufs[b].at[pl.ds(0, CHUNK_WORDS)]

    for b in range(NBUF):
        pltpu.async_copy(chunk_src(b), chunk_dst(b), sems[b])

    def compute_chunk(c, b, acc):
        buf = bufs[b]

        def row_body(j, acc):
            base = j * C
            lblv = plsc.load_gather(
                lbl, [jnp.full((L,), c * CHUNK_ROWS + j, jnp.int32)]
            )
            addr = lblv + base
            fy = plsc.load_gather(buf, [addr])
            plsc.store_scatter(
                buf, [addr], jnp.full((L,), -1e10, jnp.float32), mask=lane < 1
            )
            vals = [buf[pl.ds(base + k * L, L)] for k in range(NVREG)]
            vals.append(
                jnp.where(lane < CTAIL, buf[pl.ds(base + NVREG * L, L)], ninf)
            )
            while len(vals) > 1:
                nxt = [
                    jnp.maximum(a, b2) for a, b2 in zip(vals[0::2], vals[1::2])
                ]
                if len(vals) % 2:
                    nxt.append(vals[-1])
                vals = nxt
            fnym = jnp.max(vals[0])
            loss0 = jnp.maximum(2.0 - fy, 0.0) + jnp.maximum(1.0 + fnym, 0.0)
            return acc + jnp.where(lane < 1, loss0, zero)

        return lax.fori_loop(0, CHUNK_ROWS, row_body, acc)

    def ring_body(g, acc):
        for b in range(NBUF):
            c = g * NBUF + b
            pltpu.make_async_copy(chunk_src(c), chunk_dst(b), sems[b]).wait()
            acc = compute_chunk(c, b, acc)

            @pl.when(g < NCHUNKS // NBUF - 1)
            def _():
                pltpu.async_copy(chunk_src(c + NBUF), chunk_dst(b), sems[b])

        return acc

    acc = lax.fori_loop(
        0, NCHUNKS // NBUF, ring_body, jnp.zeros((L,), jnp.float32)
    )
    res[...] = acc
    pltpu.sync_copy(res, out_hbm.at[wid])


def kernel(prediction, label):
    pred_flat = prediction.reshape(-1)
    partials = _loss_partials(pred_flat, label)
    return jnp.sum(partials) / jnp.float32(R)


# speedup vs baseline: 7.6505x; 3.8040x over previous
import functools

import jax
import jax.numpy as jnp
from jax import lax
from jax.experimental import pallas as pl
from jax.experimental.pallas import tpu as pltpu
from jax.experimental.pallas import tpu_sc as plsc

NC = 2
NS = 16
NW = NC * NS
L = 16

S = 8
B = 4096
C = 1000
RT = S * C
CPW = B // NW
G = CPW // L
CHUNK = 200
CPS = C // CHUNK
NCHUNKS = S * CPS
NBUF = 2
RB = CHUNK // 8

_mesh = plsc.VectorSubcoreMesh(
    core_axis_name="c", subcore_axis_name="s", num_cores=NC, num_subcores=NS
)


@functools.partial(
    pl.kernel,
    out_type=jax.ShapeDtypeStruct((NW, L), jnp.float32),
    mesh=_mesh,
    compiler_params=pltpu.CompilerParams(needs_layout_passes=False),
    scratch_types=[
        pltpu.VMEM((CHUNK, CPW), jnp.float32),
        pltpu.VMEM((CHUNK, CPW), jnp.float32),
        pltpu.VMEM((CPW,), jnp.int32),
        pltpu.VMEM((L,), jnp.float32),
        pltpu.SemaphoreType.DMA,
        pltpu.SemaphoreType.DMA,
    ],
)
def _loss_partials(
    pred_hbm, label_hbm, out_hbm, buf0, buf1, lbl, res, sem0, sem1
):
    cid = lax.axis_index("c")
    sid = lax.axis_index("s")
    wid = sid * NC + cid
    col0 = wid * CPW
    pltpu.sync_copy(label_hbm.at[pl.ds(col0, CPW)], lbl)

    lane = lax.iota(jnp.int32, L)
    ninf = jnp.full((L,), -jnp.inf, jnp.float32)
    zero = jnp.zeros((L,), jnp.float32)
    sems = (sem0, sem1)
    bufs = (buf0, buf1)

    def chunk_src(c):
        return pred_hbm.at[pl.ds(c * CHUNK, CHUNK), pl.ds(col0, CPW)]

    for b in range(NBUF):
        pltpu.async_copy(chunk_src(b), bufs[b], sems[b])

    def ring_body(g2, carry):
        acc, ms, fys = carry
        for b in range(NBUF):
            c = g2 * NBUF + b
            pltpu.make_async_copy(chunk_src(c), bufs[b], sems[b]).wait()
            buf = bufs[b]
            base = lax.rem(c, CPS) * CHUNK

            ms = list(ms)
            fys = list(fys)
            for g in range(G):
                lblg = lbl[pl.ds(g * L, L)]
                li = lblg - base
                valid = (li >= 0) & (li < CHUNK)
                lic = jnp.clip(li, 0, CHUNK - 1)
                colg = lane + g * L
                got = plsc.load_gather(buf, [lic, colg], mask=valid)
                fys[g] = jnp.where(valid, got, fys[g])
                plsc.store_scatter(
                    buf, [lic, colg],
                    jnp.full((L,), -1e10, jnp.float32), mask=valid,
                )

            def rb_body(rb, ms_t):
                out = list(ms_t)
                for rr in range(8):
                    r = rb * 8 + rr
                    for g in range(G):
                        out[g] = jnp.maximum(out[g], buf[r, pl.ds(g * L, L)])
                return tuple(out)

            ms = lax.fori_loop(0, RB, rb_body, tuple(ms))

            done = jnp.broadcast_to(lax.rem(c, CPS) == CPS - 1, (L,))
            loss_sum = zero
            for g in range(G):
                loss_sum = (
                    loss_sum
                    + jnp.maximum(2.0 - fys[g], 0.0)
                    + jnp.maximum(1.0 + ms[g], 0.0)
                )
            acc = jnp.where(done, acc + loss_sum, acc)
            ms = tuple(jnp.where(done, ninf, m) for m in ms)
            fys = tuple(jnp.where(done, zero, f) for f in fys)

            @pl.when(c + NBUF < NCHUNKS)
            def _():
                pltpu.async_copy(chunk_src(c + NBUF), bufs[b], sems[b])

        return acc, ms, fys

    init = (zero, (ninf,) * G, (zero,) * G)
    acc, _, _ = lax.fori_loop(0, NCHUNKS // NBUF, ring_body, init)
    res[...] = acc
    pltpu.sync_copy(res, out_hbm.at[wid])


def kernel(prediction, label):
    pred_t = prediction.reshape(S, B, C).transpose(0, 2, 1).reshape(RT, B)
    partials = _loss_partials(pred_t, label)
    return jnp.sum(partials) / jnp.float32(S * B)


# speedup vs baseline: 8.9520x; 1.1701x over previous
import functools

import jax
import jax.numpy as jnp
from jax import lax
from jax.experimental import pallas as pl
from jax.experimental.pallas import tpu as pltpu
from jax.experimental.pallas import tpu_sc as plsc

NC = 2
NS = 16
NW = NC * NS
L = 16

S = 8
B = 4096
C = 1000
RT = S * C
CPW = B // NW
G = CPW // L
CHUNK = 200
CPS = C // CHUNK
NCHUNKS = S * CPS
NBUF = 4
RB = CHUNK // 8

_mesh = plsc.VectorSubcoreMesh(
    core_axis_name="c", subcore_axis_name="s", num_cores=NC, num_subcores=NS
)


@functools.partial(
    pl.kernel,
    out_type=jax.ShapeDtypeStruct((NW, L), jnp.float32),
    mesh=_mesh,
    compiler_params=pltpu.CompilerParams(needs_layout_passes=False),
    scratch_types=[
        pltpu.VMEM((CHUNK, CPW), jnp.float32),
        pltpu.VMEM((CHUNK, CPW), jnp.float32),
        pltpu.VMEM((CHUNK, CPW), jnp.float32),
        pltpu.VMEM((CHUNK, CPW), jnp.float32),
        pltpu.VMEM((CPW,), jnp.int32),
        pltpu.VMEM((L,), jnp.float32),
        pltpu.SemaphoreType.DMA,
        pltpu.SemaphoreType.DMA,
        pltpu.SemaphoreType.DMA,
        pltpu.SemaphoreType.DMA,
    ],
)
def _loss_partials(
    pred_hbm, label_hbm, out_hbm, buf0, buf1, buf2, buf3, lbl, res,
    sem0, sem1, sem2, sem3,
):
    cid = lax.axis_index("c")
    sid = lax.axis_index("s")
    wid = sid * NC + cid
    col0 = wid * CPW
    pltpu.sync_copy(label_hbm.at[pl.ds(col0, CPW)], lbl)

    lane = lax.iota(jnp.int32, L)
    ninf = jnp.full((L,), -jnp.inf, jnp.float32)
    zero = jnp.zeros((L,), jnp.float32)
    sems = (sem0, sem1, sem2, sem3)
    bufs = (buf0, buf1, buf2, buf3)

    def chunk_src(c):
        return pred_hbm.at[pl.ds(c * CHUNK, CHUNK), pl.ds(col0, CPW)]

    for b in range(NBUF):
        pltpu.async_copy(chunk_src(b), bufs[b], sems[b])

    def ring_body(g2, carry):
        acc, ms, fys = carry
        for b in range(NBUF):
            c = g2 * NBUF + b
            pltpu.make_async_copy(chunk_src(c), bufs[b], sems[b]).wait()
            buf = bufs[b]
            base = lax.rem(c, CPS) * CHUNK

            ms = list(ms)
            fys = list(fys)
            for g in range(G):
                lblg = lbl[pl.ds(g * L, L)]
                li = lblg - base
                valid = (li >= 0) & (li < CHUNK)
                lic = jnp.clip(li, 0, CHUNK - 1)
                colg = lane + g * L
                got = plsc.load_gather(buf, [lic, colg], mask=valid)
                fys[g] = jnp.where(valid, got, fys[g])
                plsc.store_scatter(
                    buf, [lic, colg],
                    jnp.full((L,), -1e10, jnp.float32), mask=valid,
                )

            def rb_body(rb, ms_t):
                out = list(ms_t)
                for rr in range(8):
                    r = rb * 8 + rr
                    for g in range(G):
                        out[g] = jnp.maximum(out[g], buf[r, pl.ds(g * L, L)])
                return tuple(out)

            ms = lax.fori_loop(0, RB, rb_body, tuple(ms))

            done = jnp.broadcast_to(lax.rem(c, CPS) == CPS - 1, (L,))
            loss_sum = zero
            for g in range(G):
                loss_sum = (
                    loss_sum
                    + jnp.maximum(2.0 - fys[g], 0.0)
                    + jnp.maximum(1.0 + ms[g], 0.0)
                )
            acc = jnp.where(done, acc + loss_sum, acc)
            ms = tuple(jnp.where(done, ninf, m) for m in ms)
            fys = tuple(jnp.where(done, zero, f) for f in fys)

            @pl.when(c + NBUF < NCHUNKS)
            def _():
                pltpu.async_copy(chunk_src(c + NBUF), bufs[b], sems[b])

        return acc, ms, fys

    init = (zero, (ninf,) * G, (zero,) * G)
    acc, _, _ = lax.fori_loop(0, NCHUNKS // NBUF, ring_body, init)
    res[...] = acc
    pltpu.sync_copy(res, out_hbm.at[wid])


def kernel(prediction, label):
    pred_t = prediction.reshape(S, B, C).transpose(0, 2, 1).reshape(RT, B)
    partials = _loss_partials(pred_t, label)
    return jnp.sum(partials) / jnp.float32(S * B)
